# baseline (speedup 1.0000x reference)
import jax
import jax.numpy as jnp
from jax.experimental import pallas as pl
from jax.experimental.pallas import tpu as pltpu

N, L, C = 8, 576, 768
NPAIRS = N * (N - 1) // 2
_INF = float("inf")


def _pair_ij(k):
    i = ((k >= 7).astype(jnp.int32) + (k >= 13) + (k >= 18)
         + (k >= 22) + (k >= 25) + (k >= 27))
    start = i * (N - 1) - i * (i - 1) // 2
    return i, k - start + i + 1


def _pair_index(i, j):
    return i * (N - 1) - i * (i - 1) // 2 + (j - i - 1)


def _pairs_kernel(z_ref, t_ref, u_ref,
                  zb_ref, b2h_ref, a2h_ref, b2hb_ref, a2hb_ref,
                  h0_ref, h1_ref):
    k = pl.program_id(0)

    @pl.when(k == 0)
    def _prologue():
        ones = jnp.ones((1, C), jnp.float32)
        for r in range(N):
            z = z_ref[r]
            zb_ref[r] = z.astype(jnp.bfloat16)
            sq = 0.5 * (z * z)
            b2h = jnp.sum(sq, axis=1, keepdims=True)
            b2h_ref[r] = b2h
            b2hb_ref[r] = b2h.astype(jnp.bfloat16)
            a2h = jax.lax.dot_general(
                ones, sq, (((1,), (1,)), ((), ())),
                preferred_element_type=jnp.float32)
            a2h_ref[r] = a2h
            a2hb_ref[r] = a2h.astype(jnp.bfloat16)
        h0_ref[...] = jnp.full((L, L), -_INF, jnp.bfloat16)
        h1_ref[...] = jnp.full((L, L), -_INF, jnp.bfloat16)

    i_d, j_d = _pair_ij(jnp.minimum(k, NPAIRS - 1))
    i_e, j_e = _pair_ij(jnp.maximum(k - 1, 0))

    def _dot(h_ref):
        h_ref[...] = jax.lax.dot_general(
            zb_ref[j_d], zb_ref[i_d], (((1,), (1,)), ((), ())),
            preferred_element_type=jnp.float32).astype(jnp.bfloat16)

    def _epilogue(h_ref):
        h = h_ref[...]
        t = jnp.min(b2hb_ref[j_e] - h, axis=0, keepdims=True)
        t_ref[0] = jnp.maximum(
            2.0 * (a2h_ref[i_e] + t.astype(jnp.float32)), 0.0)
        u = jnp.min(a2hb_ref[i_e] - h, axis=1, keepdims=True)
        u_ref[0] = jnp.maximum(
            2.0 * (b2h_ref[j_e] + u.astype(jnp.float32)), 0.0)

    @pl.when(k % 2 == 0)
    def _even():
        _dot(h0_ref)
        _epilogue(h1_ref)

    @pl.when(k % 2 == 1)
    def _odd():
        _dot(h1_ref)
        _epilogue(h0_ref)


def _select_kernel(t_ref, u_ref, out_ref):
    t = t_ref[...]
    u = u_ref[...]
    rows = []
    for r in range(N):
        cand = [u[_pair_index(i, r)] for i in range(r)]
        cand += [t[_pair_index(r, j)] for j in range(r + 1, N)]
        m1 = jnp.minimum(cand[0], cand[1])
        m2 = jnp.maximum(cand[0], cand[1])
        for c in cand[2:]:
            m2 = jnp.minimum(m2, jnp.maximum(m1, c))
            m1 = jnp.minimum(m1, c)
        rows.append(0.5 * (jnp.sqrt(m1) + jnp.sqrt(m2)))
    out_ref[...] = jnp.stack(rows, axis=0)


@jax.jit
def kernel(Z):
    full = lambda s: pl.BlockSpec(s, lambda k: (0,) * len(s))
    lag = lambda ndim: (lambda k: (jnp.maximum(k - 1, 0),) + (0,) * (ndim - 1))
    t_st, u_st = pl.pallas_call(
        _pairs_kernel,
        grid=(NPAIRS + 1,),
        in_specs=[full((N, L, C))],
        out_specs=[pl.BlockSpec((1, 1, L), lag(3)),
                   pl.BlockSpec((1, L, 1), lag(3))],
        out_shape=[jax.ShapeDtypeStruct((NPAIRS, 1, L), jnp.float32),
                   jax.ShapeDtypeStruct((NPAIRS, L, 1), jnp.float32)],
        scratch_shapes=[
            pltpu.VMEM((N, L, C), jnp.bfloat16),
            pltpu.VMEM((N, L, 1), jnp.float32),
            pltpu.VMEM((N, 1, L), jnp.float32),
            pltpu.VMEM((N, L, 1), jnp.bfloat16),
            pltpu.VMEM((N, 1, L), jnp.bfloat16),
            pltpu.VMEM((L, L), jnp.bfloat16),
            pltpu.VMEM((L, L), jnp.bfloat16),
        ],
    )(Z)
    args = (t_st.reshape(NPAIRS, L), u_st.reshape(NPAIRS, L))
    spec = pl.BlockSpec((NPAIRS, L), lambda: (0, 0))
    return pl.pallas_call(
        _select_kernel,
        in_specs=[spec] * 2,
        out_specs=pl.BlockSpec((N, L), lambda: (0, 0)),
        out_shape=jax.ShapeDtypeStruct((N, L), jnp.float32),
    )(*args)


# speedup vs baseline: 1.1460x; 1.1460x over previous
import jax
import jax.numpy as jnp
from jax.experimental import pallas as pl
from jax.experimental.pallas import tpu as pltpu

N, L, C = 8, 576, 768
NPAIRS = N * (N - 1) // 2
_INF = float("inf")


def _pair_ij(k):
    i = ((k >= 7).astype(jnp.int32) + (k >= 13) + (k >= 18)
         + (k >= 22) + (k >= 25) + (k >= 27))
    start = i * (N - 1) - i * (i - 1) // 2
    return i, k - start + i + 1


def _pairs_kernel(z_ref, out_ref,
                  zb_ref, b2h_ref, a2h_ref, b2hb_ref, a2hb_ref,
                  rm1_ref, rm2_ref, cm1_ref, cm2_ref,
                  eye_ref, h0_ref, h1_ref):
    k = pl.program_id(0)

    @pl.when(k == 0)
    def _prologue():
        ones = jnp.ones((1, C), jnp.float32)
        for r in range(N):
            z = z_ref[r]
            zb_ref[r] = z.astype(jnp.bfloat16)
            sq = 0.5 * (z * z)
            b2h = jnp.sum(sq, axis=1, keepdims=True)
            b2h_ref[r] = b2h
            b2hb_ref[r] = b2h.astype(jnp.bfloat16)
            a2h = jax.lax.dot_general(
                ones, sq, (((1,), (1,)), ((), ())),
                preferred_element_type=jnp.float32)
            a2h_ref[r] = a2h
            a2hb_ref[r] = a2h.astype(jnp.bfloat16)
        rm1_ref[...] = jnp.full((N, 1, L), _INF, jnp.float32)
        rm2_ref[...] = jnp.full((N, 1, L), _INF, jnp.float32)
        cm1_ref[...] = jnp.full((N, L, 1), _INF, jnp.float32)
        cm2_ref[...] = jnp.full((N, L, 1), _INF, jnp.float32)
        rows = jax.lax.broadcasted_iota(jnp.int32, (L, L), 0)
        cols = jax.lax.broadcasted_iota(jnp.int32, (L, L), 1)
        eye_ref[...] = jnp.where(rows == cols, 1.0, 0.0).astype(jnp.bfloat16)
        h0_ref[...] = jnp.full((L, L), -_INF, jnp.bfloat16)
        h1_ref[...] = jnp.full((L, L), -_INF, jnp.bfloat16)

    i_d, j_d = _pair_ij(jnp.minimum(k, NPAIRS - 1))
    i_e, j_e = _pair_ij(jnp.maximum(k - 1, 0))

    def _dot(h_ref):
        h_ref[...] = jax.lax.dot_general(
            zb_ref[j_d], zb_ref[i_d], (((1,), (1,)), ((), ())),
            preferred_element_type=jnp.float32).astype(jnp.bfloat16)

    def _epilogue(h_ref):
        h = h_ref[...]
        t = jnp.min(b2hb_ref[j_e] - h, axis=0, keepdims=True)
        vi = jnp.maximum(2.0 * (a2h_ref[i_e] + t.astype(jnp.float32)), 0.0)
        m1 = rm1_ref[i_e]
        m2 = rm2_ref[i_e]
        rm1_ref[i_e] = jnp.minimum(m1, vi)
        rm2_ref[i_e] = jnp.minimum(m2, jnp.maximum(m1, vi))
        u = jnp.min(a2hb_ref[i_e] - h, axis=1, keepdims=True)
        vj = jnp.maximum(2.0 * (b2h_ref[j_e] + u.astype(jnp.float32)), 0.0)
        m1 = cm1_ref[j_e]
        m2 = cm2_ref[j_e]
        cm1_ref[j_e] = jnp.minimum(m1, vj)
        cm2_ref[j_e] = jnp.minimum(m2, jnp.maximum(m1, vj))

    @pl.when(k % 2 == 0)
    def _even():
        _dot(h0_ref)
        _epilogue(h1_ref)

    @pl.when(k % 2 == 1)
    def _odd():
        _dot(h1_ref)
        _epilogue(h0_ref)

    @pl.when(k == NPAIRS)
    def _finish():
        eye = eye_ref[...]
        for r in range(N):
            c1 = jnp.minimum(cm1_ref[r], 1e30)
            c2 = jnp.minimum(cm2_ref[r], 1e30)
            c1h = c1.astype(jnp.bfloat16)
            c2h = c2.astype(jnp.bfloat16)
            c1l = (c1 - c1h.astype(jnp.float32)).astype(jnp.bfloat16)
            c2l = (c2 - c2h.astype(jnp.float32)).astype(jnp.bfloat16)
            stacked = jnp.concatenate([c1h, c1l, c2h, c2l], axis=1)
            tr = jax.lax.dot_general(
                stacked, eye, (((0,), (0,)), ((), ())),
                preferred_element_type=jnp.float32)
            c1r = tr[0:1, :] + tr[1:2, :]
            c2r = tr[2:3, :] + tr[3:4, :]
            r1 = rm1_ref[r]
            r2 = rm2_ref[r]
            m1 = jnp.minimum(r1, c1r)
            m2 = jnp.minimum(jnp.maximum(r1, c1r), jnp.minimum(r2, c2r))
            out_ref[r] = 0.5 * (jnp.sqrt(m1) + jnp.sqrt(m2))


@jax.jit
def kernel(Z):
    full = lambda s: pl.BlockSpec(s, lambda k: (0,) * len(s))
    out = pl.pallas_call(
        _pairs_kernel,
        grid=(NPAIRS + 1,),
        in_specs=[full((N, L, C))],
        out_specs=full((N, 1, L)),
        out_shape=jax.ShapeDtypeStruct((N, 1, L), jnp.float32),
        scratch_shapes=[
            pltpu.VMEM((N, L, C), jnp.bfloat16),
            pltpu.VMEM((N, L, 1), jnp.float32),
            pltpu.VMEM((N, 1, L), jnp.float32),
            pltpu.VMEM((N, L, 1), jnp.bfloat16),
            pltpu.VMEM((N, 1, L), jnp.bfloat16),
            pltpu.VMEM((N, 1, L), jnp.float32),
            pltpu.VMEM((N, 1, L), jnp.float32),
            pltpu.VMEM((N, L, 1), jnp.float32),
            pltpu.VMEM((N, L, 1), jnp.float32),
            pltpu.VMEM((L, L), jnp.bfloat16),
            pltpu.VMEM((L, L), jnp.bfloat16),
            pltpu.VMEM((L, L), jnp.bfloat16),
        ],
    )(Z)
    return out[:, 0, :]
